# baseline (speedup 1.0000x reference)
import functools

import jax
import jax.numpy as jnp
from jax import lax
from jax.experimental import pallas as pl
from jax.experimental.pallas import tpu as pltpu
from jax.experimental.pallas import tpu_sc as plsc

F32 = jnp.float32
NC, NS, LANES = 2, 16, 16
CW = 128
IB = 8
NBUF = 8
ZR = 200
BN = 2048


_SC_PARAMS = pltpu.CompilerParams(use_tc_tiling_on_sc=False)


def _mesh():
    return plsc.VectorSubcoreMesh(core_axis_name="c", subcore_axis_name="s",
                                  num_cores=NC, num_subcores=NS)


def _fill(buf, rows, h, value):
    v = jnp.full((LANES,), value, F32)

    @pl.loop(0, rows)
    def _(r):
        for c0 in range(0, h, LANES):
            buf[r, pl.ds(c0, LANES)] = v


def _zero_acc(acc_sh, zeros_v, sid, rps):
    @pl.loop(0, rps, step=ZR)
    def _(r0):
        pltpu.sync_copy(zeros_v, acc_sh.at[pl.ds(sid * rps + r0, ZR)])


def _agg_pipe(tab, srcm, dstm, ones_v, src_i, dst_i, rows_v, acc_sh,
              sem_ix, sem_g, sem_s, first_blk, n_blocks):
    gather = tab is not None

    def idx_issue(blk, p):
        if gather:
            pltpu.async_copy(srcm.at[pl.ds(blk * IB, IB)], src_i.at[p],
                             sem_ix.at[p])
        pltpu.async_copy(dstm.at[pl.ds(blk * IB, IB)], dst_i.at[p],
                         sem_ix.at[p])

    def idx_wait(blk, p):
        if gather:
            pltpu.make_async_copy(srcm.at[pl.ds(blk * IB, IB)], src_i.at[p],
                                  sem_ix.at[p]).wait()
        pltpu.make_async_copy(dstm.at[pl.ds(blk * IB, IB)], dst_i.at[p],
                              sem_ix.at[p]).wait()

    idx_issue(first_blk, 0)

    @pl.loop(0, n_blocks)
    def _(ib):
        blk = first_blk + ib
        p = lax.rem(ib, 2)
        idx_wait(blk, p)

        def stage_a(k):
            src_ref = rows_v.at[k] if gather else ones_v

            def w():
                pltpu.make_async_copy(src_ref, acc_sh.at[dst_i.at[p, k]],
                                      sem_s.at[k]).wait()

            pl.when(ib > 0)(w)
            if gather:
                pltpu.async_copy(tab.at[src_i.at[p, k]], rows_v.at[k],
                                 sem_g.at[k])

        def stage_b(k):
            if gather:
                pltpu.make_async_copy(tab.at[src_i.at[p, k]], rows_v.at[k],
                                      sem_g.at[k]).wait()
                pltpu.async_copy(rows_v.at[k], acc_sh.at[dst_i.at[p, k]],
                                 sem_s.at[k], add=True)
            else:
                pltpu.async_copy(ones_v, acc_sh.at[dst_i.at[p, k]],
                                 sem_s.at[k], add=True)

        for k in range(IB):
            stage_a(k)
        nxt = first_blk + jnp.minimum(ib + 1, n_blocks - 1)
        idx_issue(nxt, 1 - p)
        for k in range(IB):
            stage_b(k)

    for k in range(IB):
        src_ref = rows_v.at[k] if gather else ones_v
        pltpu.make_async_copy(src_ref, acc_sh.at[dst_i.at[0, k]],
                              sem_s.at[k]).wait()
    idx_wait(first_blk, n_blocks % 2)


def _sc_count(npad, rps, blocks, dstm):
    bpw = blocks // (NC * NS)

    @functools.partial(
        pl.kernel,
        mesh=_mesh(),
        compiler_params=_SC_PARAMS,
        out_type=jax.ShapeDtypeStruct((NC, npad, 16), F32),
        scratch_types=[
            pltpu.VMEM_SHARED((npad, 16), F32),
            pltpu.VMEM((2, IB, CW), jnp.int32),
            pltpu.VMEM((CW, 16), F32),
            pltpu.VMEM((ZR, 16), F32),
            pltpu.SemaphoreType.DMA((2,)),
            pltpu.SemaphoreType.DMA((IB,)),
        ],
    )
    def k(dstm_hbm, out_hbm, acc_sh, dst_i, ones_v, zeros_v, sem_ix, sem_s):
        c = lax.axis_index("c")
        s = lax.axis_index("s")
        _fill(zeros_v, ZR, 16, 0.0)
        _fill(ones_v, CW, 16, 1.0)
        _zero_acc(acc_sh, zeros_v, s, rps)
        plsc.subcore_barrier()
        _agg_pipe(None, None, dstm_hbm, ones_v, None, dst_i, None, acc_sh,
                  sem_ix, None, sem_s, (c * NS + s) * bpw, bpw)
        plsc.subcore_barrier()
        pltpu.sync_copy(acc_sh.at[pl.ds(s * rps, rps)],
                        out_hbm.at[c, pl.ds(s * rps, rps)])

    return k(dstm)


def _sc_agg_phased(npad, rps, blocks, nphase, ys, srcm, dstm):
    bpw = blocks // (NC * NS)
    spt = npad // NS

    @functools.partial(
        pl.kernel,
        mesh=_mesh(),
        compiler_params=_SC_PARAMS,
        out_type=jax.ShapeDtypeStruct((NC, nphase, npad, 16), F32),
        scratch_types=[
            pltpu.VMEM_SHARED((npad, 16), F32),
            pltpu.VMEM_SHARED((npad, 16), F32),
            pltpu.VMEM((2, IB, CW), jnp.int32),
            pltpu.VMEM((2, IB, CW), jnp.int32),
            pltpu.VMEM((NBUF, CW, 16), F32),
            pltpu.VMEM((ZR, 16), F32),
            pltpu.SemaphoreType.DMA((2,)),
            pltpu.SemaphoreType.DMA((NBUF,)),
            pltpu.SemaphoreType.DMA((NBUF,)),
        ],
    )
    def k(ys_hbm, srcm_hbm, dstm_hbm, out_hbm,
          tab_sh, acc_sh, src_i, dst_i, rows_v, zeros_v, sem_ix, sem_g, sem_s):
        c = lax.axis_index("c")
        s = lax.axis_index("s")
        _fill(zeros_v, ZR, 16, 0.0)
        pltpu.sync_copy(ys_hbm.at[0, pl.ds(s * spt, spt)],
                        tab_sh.at[pl.ds(s * spt, spt)])
        _zero_acc(acc_sh, zeros_v, s, rps)
        plsc.subcore_barrier()
        first = (c * NS + s) * bpw

        @pl.loop(0, nphase)
        def _(ph):
            _agg_pipe(tab_sh, srcm_hbm, dstm_hbm, None, src_i, dst_i, rows_v,
                      acc_sh, sem_ix, sem_g, sem_s, first, bpw)
            plsc.subcore_barrier()
            pltpu.sync_copy(acc_sh.at[pl.ds(s * rps, rps)],
                            out_hbm.at[c, ph, pl.ds(s * rps, rps)])

            @pl.when(ph < nphase - 1)
            def _():
                pltpu.sync_copy(ys_hbm.at[ph + 1, pl.ds(s * spt, spt)],
                                tab_sh.at[pl.ds(s * spt, spt)])
                _zero_acc(acc_sh, zeros_v, s, rps)

            plsc.subcore_barrier()

    return k(ys, srcm, dstm)


_HI = jax.lax.Precision.HIGHEST


def _dot(a, b):
    return jnp.dot(a, b, precision=_HI, preferred_element_type=F32)


def _tc1(n, d, npad, x, wl1, wr1):
    def body(x_ref, wl_ref, wr_ref, y1_ref, r1_ref):
        xb = x_ref[...]
        yl = _dot(xb, wl_ref[...])
        for j in range(4):
            y1_ref[j] = yl[:, 16 * j:16 * (j + 1)]
        r1_ref[...] = _dot(xb, wr_ref[...])

    return pl.pallas_call(
        body,
        grid=(-(-n // BN),),
        in_specs=[pl.BlockSpec((BN, d), lambda i: (i, 0)),
                  pl.BlockSpec((d, 64), lambda i: (0, 0)),
                  pl.BlockSpec((d, 64), lambda i: (0, 0))],
        out_specs=[pl.BlockSpec((4, BN, 16), lambda i: (0, i, 0)),
                   pl.BlockSpec((BN, 64), lambda i: (i, 0))],
        out_shape=[jax.ShapeDtypeStruct((4, npad, 16), F32),
                   jax.ShapeDtypeStruct((n, 64), F32)],
    )(x, wl1, wr1)


def _tc_cnt(npad, rps, cntp):
    def body(cp_ref, invc_ref):
        cnt = cp_ref[0, :, 0:1] + cp_ref[1, :, 0:1]
        invc_ref[...] = 1.0 / jnp.maximum(cnt, 1.0)

    return pl.pallas_call(
        body,
        grid=(npad // rps,),
        in_specs=[pl.BlockSpec((2, rps, 16), lambda i: (0, i, 0))],
        out_specs=pl.BlockSpec((rps, 1), lambda i: (i, 0)),
        out_shape=jax.ShapeDtypeStruct((npad, 1), F32),
    )(cntp)


def _tc2(n, npad, agg1p, r1, invc, b1, wl2, wr2):
    def body(agg_ref, r1_ref, invc_ref, b1_ref, wl_ref, wr_ref, y2_ref, r2_ref):
        iv = invc_ref[...]
        mean = jnp.concatenate(
            [(agg_ref[0, j] + agg_ref[1, j]) * iv for j in range(4)], axis=1)
        h1 = jnp.maximum(mean + r1_ref[...] + b1_ref[...], 0.0)
        y2 = _dot(h1, wl_ref[...])
        for j in range(2):
            y2_ref[j] = y2[:, 16 * j:16 * (j + 1)]
        r2_ref[...] = _dot(h1, wr_ref[...])

    return pl.pallas_call(
        body,
        grid=(-(-n // BN),),
        in_specs=[pl.BlockSpec((2, 4, BN, 16), lambda i: (0, 0, i, 0)),
                  pl.BlockSpec((BN, 64), lambda i: (i, 0)),
                  pl.BlockSpec((BN, 1), lambda i: (i, 0)),
                  pl.BlockSpec((1, 64), lambda i: (0, 0)),
                  pl.BlockSpec((64, 32), lambda i: (0, 0)),
                  pl.BlockSpec((64, 32), lambda i: (0, 0))],
        out_specs=[pl.BlockSpec((2, BN, 16), lambda i: (0, i, 0)),
                   pl.BlockSpec((BN, 32), lambda i: (i, 0))],
        out_shape=[jax.ShapeDtypeStruct((2, npad, 16), F32),
                   jax.ShapeDtypeStruct((n, 32), F32)],
    )(agg1p, r1, invc, b1.reshape(1, 64), wl2, wr2)


def _tc3(n, npad, agg2p, r2, invc, b2, wl3, wr3, b3):
    def body(agg_ref, r2_ref, invc_ref, b2_ref, wl_ref, wr_ref, b3_ref,
             y3b_ref, r3_ref):
        iv = invc_ref[...]
        mean = jnp.concatenate(
            [(agg_ref[0, j] + agg_ref[1, j]) * iv for j in range(2)], axis=1)
        h2 = jnp.maximum(mean + r2_ref[...] + b2_ref[...], 0.0)
        y3 = _dot(h2, wl_ref[...])
        y3b_ref[...] = jnp.broadcast_to(y3, (BN, 16)).reshape(1, BN, 16)
        r3_ref[...] = _dot(h2, wr_ref[...]) + b3_ref[...]

    return pl.pallas_call(
        body,
        grid=(-(-n // BN),),
        in_specs=[pl.BlockSpec((2, 2, BN, 16), lambda i: (0, 0, i, 0)),
                  pl.BlockSpec((BN, 32), lambda i: (i, 0)),
                  pl.BlockSpec((BN, 1), lambda i: (i, 0)),
                  pl.BlockSpec((1, 32), lambda i: (0, 0)),
                  pl.BlockSpec((32, 1), lambda i: (0, 0)),
                  pl.BlockSpec((32, 1), lambda i: (0, 0)),
                  pl.BlockSpec((1, 1), lambda i: (0, 0))],
        out_specs=[pl.BlockSpec((1, BN, 16), lambda i: (0, i, 0)),
                   pl.BlockSpec((BN, 1), lambda i: (i, 0))],
        out_shape=[jax.ShapeDtypeStruct((1, npad, 16), F32),
                   jax.ShapeDtypeStruct((n, 1), F32)],
    )(agg2p, r2, invc, b2.reshape(1, 32), wl3, wr3, b3.reshape(1, 1))


def _tc4(n, agg3p, invc, r3):
    def body(a3_ref, invc_ref, r3_ref, o_ref):
        s3 = a3_ref[0, 0, :, 0:1] + a3_ref[1, 0, :, 0:1]
        o_ref[...] = jax.nn.sigmoid(s3 * invc_ref[...] + r3_ref[...])

    return pl.pallas_call(
        body,
        grid=(-(-n // BN),),
        in_specs=[pl.BlockSpec((2, 1, BN, 16), lambda i: (0, 0, i, 0)),
                  pl.BlockSpec((BN, 1), lambda i: (i, 0)),
                  pl.BlockSpec((BN, 1), lambda i: (i, 0))],
        out_specs=pl.BlockSpec((BN, 1), lambda i: (i, 0)),
        out_shape=jax.ShapeDtypeStruct((n, 1), F32),
    )(agg3p, invc, r3)


def kernel(x, edge_index, Wl1, Wr1, b1, Wl2, Wr2, b2, Wl3, Wr3, b3):
    n, d = x.shape
    e = edge_index.shape[1]

    grain = NS * ZR * 8
    npad = -(-(n + 1) // grain) * grain
    rps = npad // NS

    epg = NC * NS * CW * IB
    ep = -(-e // epg) * epg
    src = edge_index[0].astype(jnp.int32)
    dst = edge_index[1].astype(jnp.int32)
    trash = n + jnp.arange(ep - e, dtype=jnp.int32) % (npad - n)
    srcm = jnp.concatenate([src, jnp.zeros((ep - e,), jnp.int32)]).reshape(ep // CW, CW)
    dstm = jnp.concatenate([dst, trash]).reshape(ep // CW, CW)
    blocks = ep // (CW * IB)

    cntp = _sc_count(npad, rps, blocks, dstm)
    y1s, r1 = _tc1(n, d, npad, x, Wl1, Wr1)
    agg1p = _sc_agg_phased(npad, rps, blocks, 4, y1s, srcm, dstm)
    invc = _tc_cnt(npad, rps, cntp)
    y2s, r2 = _tc2(n, npad, agg1p, r1, invc, b1, Wl2, Wr2)
    agg2p = _sc_agg_phased(npad, rps, blocks, 2, y2s, srcm, dstm)
    y3s, r3 = _tc3(n, npad, agg2p, r2, invc, b2, Wl3, Wr3, b3)
    agg3p = _sc_agg_phased(npad, rps, blocks, 1, y3s, srcm, dstm)
    return _tc4(n, agg3p, invc, r3)


# speedup vs baseline: 1.0231x; 1.0231x over previous
import functools

import jax
import jax.numpy as jnp
from jax import lax
from jax.experimental import pallas as pl
from jax.experimental.pallas import tpu as pltpu
from jax.experimental.pallas import tpu_sc as plsc

F32 = jnp.float32
NC, NS, LANES = 2, 16, 16
CW = 128
IB = 8
NBUF = 8
ZR = 200
BN = 2048


_SC_PARAMS = pltpu.CompilerParams(use_tc_tiling_on_sc=False)


def _mesh():
    return plsc.VectorSubcoreMesh(core_axis_name="c", subcore_axis_name="s",
                                  num_cores=NC, num_subcores=NS)


def _fill(buf, rows, h, value):
    v = jnp.full((LANES,), value, F32)

    @pl.loop(0, rows)
    def _(r):
        for c0 in range(0, h, LANES):
            buf[r, pl.ds(c0, LANES)] = v


def _zero_acc(acc_sh, zeros_v, sid, rps):
    @pl.loop(0, rps, step=ZR)
    def _(r0):
        pltpu.sync_copy(zeros_v, acc_sh.at[pl.ds(sid * rps + r0, ZR)])


def _agg_pipe(tab, srcm, dstm, ones_v, src_i, dst_i, rows_v, acc_sh,
              sem_ix, sem_g, sem_s, first_blk, n_blocks):
    gather = tab is not None

    def idx_issue(blk, p):
        if gather:
            pltpu.async_copy(srcm.at[pl.ds(blk * IB, IB)], src_i.at[p],
                             sem_ix.at[p])
        pltpu.async_copy(dstm.at[pl.ds(blk * IB, IB)], dst_i.at[p],
                         sem_ix.at[p])

    def idx_wait(blk, p):
        if gather:
            pltpu.make_async_copy(srcm.at[pl.ds(blk * IB, IB)], src_i.at[p],
                                  sem_ix.at[p]).wait()
        pltpu.make_async_copy(dstm.at[pl.ds(blk * IB, IB)], dst_i.at[p],
                              sem_ix.at[p]).wait()

    idx_issue(first_blk, 0)

    @pl.loop(0, n_blocks)
    def _(ib):
        blk = first_blk + ib
        p = lax.rem(ib, 2)
        idx_wait(blk, p)

        def stage_a(k):
            src_ref = rows_v.at[k] if gather else ones_v

            def w():
                pltpu.make_async_copy(src_ref, acc_sh.at[dst_i.at[p, k]],
                                      sem_s.at[k]).wait()

            pl.when(ib > 0)(w)
            if gather:
                pltpu.async_copy(tab.at[src_i.at[p, k]], rows_v.at[k],
                                 sem_g.at[k])

        def stage_b(k):
            if gather:
                pltpu.make_async_copy(tab.at[src_i.at[p, k]], rows_v.at[k],
                                      sem_g.at[k]).wait()
                pltpu.async_copy(rows_v.at[k], acc_sh.at[dst_i.at[p, k]],
                                 sem_s.at[k], add=True)
            else:
                pltpu.async_copy(ones_v, acc_sh.at[dst_i.at[p, k]],
                                 sem_s.at[k], add=True)

        for k in range(IB):
            stage_a(k)
        nxt = first_blk + jnp.minimum(ib + 1, n_blocks - 1)
        idx_issue(nxt, 1 - p)
        for k in range(IB):
            stage_b(k)

    for k in range(IB):
        src_ref = rows_v.at[k] if gather else ones_v
        pltpu.make_async_copy(src_ref, acc_sh.at[dst_i.at[0, k]],
                              sem_s.at[k]).wait()
    idx_wait(first_blk, n_blocks % 2)


def _sc_count(npad, rps, blocks, dstm):
    bpw = blocks // (NC * NS)

    @functools.partial(
        pl.kernel,
        mesh=_mesh(),
        compiler_params=_SC_PARAMS,
        out_type=jax.ShapeDtypeStruct((NC, npad, 16), F32),
        scratch_types=[
            pltpu.VMEM_SHARED((npad, 16), F32),
            pltpu.VMEM((2, IB, CW), jnp.int32),
            pltpu.VMEM((CW, 16), F32),
            pltpu.VMEM((ZR, 16), F32),
            pltpu.SemaphoreType.DMA((2,)),
            pltpu.SemaphoreType.DMA((IB,)),
        ],
    )
    def k(dstm_hbm, out_hbm, acc_sh, dst_i, ones_v, zeros_v, sem_ix, sem_s):
        c = lax.axis_index("c")
        s = lax.axis_index("s")
        _fill(zeros_v, ZR, 16, 0.0)
        _fill(ones_v, CW, 16, 1.0)
        _zero_acc(acc_sh, zeros_v, s, rps)
        plsc.subcore_barrier()
        _agg_pipe(None, None, dstm_hbm, ones_v, None, dst_i, None, acc_sh,
                  sem_ix, None, sem_s, (c * NS + s) * bpw, bpw)
        plsc.subcore_barrier()
        pltpu.sync_copy(acc_sh.at[pl.ds(s * rps, rps)],
                        out_hbm.at[c, pl.ds(s * rps, rps)])

    return k(dstm)


def _sc_agg_phased(npad, rps, blocks, nphase, ys, srcm, dstm):
    bpw = blocks // (NC * NS)
    spt = npad // NS

    @functools.partial(
        pl.kernel,
        mesh=_mesh(),
        compiler_params=_SC_PARAMS,
        out_type=jax.ShapeDtypeStruct((NC, nphase, npad, 16), F32),
        scratch_types=[
            pltpu.VMEM_SHARED((npad, 16), F32),
            pltpu.VMEM_SHARED((npad, 16), F32),
            pltpu.VMEM((2, IB, CW), jnp.int32),
            pltpu.VMEM((2, IB, CW), jnp.int32),
            pltpu.VMEM((NBUF, CW, 16), F32),
            pltpu.VMEM((ZR, 16), F32),
            pltpu.SemaphoreType.DMA((2,)),
            pltpu.SemaphoreType.DMA((NBUF,)),
            pltpu.SemaphoreType.DMA((NBUF,)),
        ],
    )
    def k(ys_hbm, srcm_hbm, dstm_hbm, out_hbm,
          tab_sh, acc_sh, src_i, dst_i, rows_v, zeros_v, sem_ix, sem_g, sem_s):
        c = lax.axis_index("c")
        s = lax.axis_index("s")
        _fill(zeros_v, ZR, 16, 0.0)
        pltpu.sync_copy(ys_hbm.at[0, pl.ds(s * spt, spt)],
                        tab_sh.at[pl.ds(s * spt, spt)])
        _zero_acc(acc_sh, zeros_v, s, rps)
        plsc.subcore_barrier()
        first = (c * NS + s) * bpw

        @pl.loop(0, nphase)
        def _(ph):
            _agg_pipe(tab_sh, srcm_hbm, dstm_hbm, None, src_i, dst_i, rows_v,
                      acc_sh, sem_ix, sem_g, sem_s, first, bpw)
            plsc.subcore_barrier()
            pltpu.sync_copy(acc_sh.at[pl.ds(s * rps, rps)],
                            out_hbm.at[c, ph, pl.ds(s * rps, rps)])

            @pl.when(ph < nphase - 1)
            def _():
                pltpu.sync_copy(ys_hbm.at[ph + 1, pl.ds(s * spt, spt)],
                                tab_sh.at[pl.ds(s * spt, spt)])
                _zero_acc(acc_sh, zeros_v, s, rps)

            plsc.subcore_barrier()

    return k(ys, srcm, dstm)


def _dot(a, b):
    return jnp.dot(a, b, precision=jax.lax.Precision.DEFAULT,
                   preferred_element_type=F32)


def _tc1(n, d, npad, x, wl1, wr1):
    def body(x_ref, wl_ref, wr_ref, y1_ref, r1_ref):
        xb = x_ref[...]
        yl = _dot(xb, wl_ref[...])
        for j in range(4):
            y1_ref[j] = yl[:, 16 * j:16 * (j + 1)]
        r1_ref[...] = _dot(xb, wr_ref[...])

    return pl.pallas_call(
        body,
        grid=(-(-n // BN),),
        in_specs=[pl.BlockSpec((BN, d), lambda i: (i, 0)),
                  pl.BlockSpec((d, 64), lambda i: (0, 0)),
                  pl.BlockSpec((d, 64), lambda i: (0, 0))],
        out_specs=[pl.BlockSpec((4, BN, 16), lambda i: (0, i, 0)),
                   pl.BlockSpec((BN, 64), lambda i: (i, 0))],
        out_shape=[jax.ShapeDtypeStruct((4, npad, 16), F32),
                   jax.ShapeDtypeStruct((n, 64), F32)],
    )(x, wl1, wr1)


def _tc_cnt(npad, rps, cntp):
    def body(cp_ref, invc_ref):
        cnt = cp_ref[0, :, 0:1] + cp_ref[1, :, 0:1]
        invc_ref[...] = 1.0 / jnp.maximum(cnt, 1.0)

    return pl.pallas_call(
        body,
        grid=(npad // rps,),
        in_specs=[pl.BlockSpec((2, rps, 16), lambda i: (0, i, 0))],
        out_specs=pl.BlockSpec((rps, 1), lambda i: (i, 0)),
        out_shape=jax.ShapeDtypeStruct((npad, 1), F32),
    )(cntp)


def _tc2(n, npad, agg1p, r1, invc, b1, wl2, wr2):
    def body(agg_ref, r1_ref, invc_ref, b1_ref, wl_ref, wr_ref, y2_ref, r2_ref):
        iv = invc_ref[...]
        mean = jnp.concatenate(
            [(agg_ref[0, j] + agg_ref[1, j]) * iv for j in range(4)], axis=1)
        h1 = jnp.maximum(mean + r1_ref[...] + b1_ref[...], 0.0)
        y2 = _dot(h1, wl_ref[...])
        for j in range(2):
            y2_ref[j] = y2[:, 16 * j:16 * (j + 1)]
        r2_ref[...] = _dot(h1, wr_ref[...])

    return pl.pallas_call(
        body,
        grid=(-(-n // BN),),
        in_specs=[pl.BlockSpec((2, 4, BN, 16), lambda i: (0, 0, i, 0)),
                  pl.BlockSpec((BN, 64), lambda i: (i, 0)),
                  pl.BlockSpec((BN, 1), lambda i: (i, 0)),
                  pl.BlockSpec((1, 64), lambda i: (0, 0)),
                  pl.BlockSpec((64, 32), lambda i: (0, 0)),
                  pl.BlockSpec((64, 32), lambda i: (0, 0))],
        out_specs=[pl.BlockSpec((2, BN, 16), lambda i: (0, i, 0)),
                   pl.BlockSpec((BN, 32), lambda i: (i, 0))],
        out_shape=[jax.ShapeDtypeStruct((2, npad, 16), F32),
                   jax.ShapeDtypeStruct((n, 32), F32)],
    )(agg1p, r1, invc, b1.reshape(1, 64), wl2, wr2)


def _tc3(n, npad, agg2p, r2, invc, b2, wl3, wr3, b3):
    def body(agg_ref, r2_ref, invc_ref, b2_ref, wl_ref, wr_ref, b3_ref,
             y3b_ref, r3_ref):
        iv = invc_ref[...]
        mean = jnp.concatenate(
            [(agg_ref[0, j] + agg_ref[1, j]) * iv for j in range(2)], axis=1)
        h2 = jnp.maximum(mean + r2_ref[...] + b2_ref[...], 0.0)
        y3 = _dot(h2, wl_ref[...])
        y3b_ref[...] = jnp.broadcast_to(y3, (BN, 16)).reshape(1, BN, 16)
        r3_ref[...] = _dot(h2, wr_ref[...]) + b3_ref[...]

    return pl.pallas_call(
        body,
        grid=(-(-n // BN),),
        in_specs=[pl.BlockSpec((2, 2, BN, 16), lambda i: (0, 0, i, 0)),
                  pl.BlockSpec((BN, 32), lambda i: (i, 0)),
                  pl.BlockSpec((BN, 1), lambda i: (i, 0)),
                  pl.BlockSpec((1, 32), lambda i: (0, 0)),
                  pl.BlockSpec((32, 1), lambda i: (0, 0)),
                  pl.BlockSpec((32, 1), lambda i: (0, 0)),
                  pl.BlockSpec((1, 1), lambda i: (0, 0))],
        out_specs=[pl.BlockSpec((1, BN, 16), lambda i: (0, i, 0)),
                   pl.BlockSpec((BN, 1), lambda i: (i, 0))],
        out_shape=[jax.ShapeDtypeStruct((1, npad, 16), F32),
                   jax.ShapeDtypeStruct((n, 1), F32)],
    )(agg2p, r2, invc, b2.reshape(1, 32), wl3, wr3, b3.reshape(1, 1))


def _tc4(n, agg3p, invc, r3):
    def body(a3_ref, invc_ref, r3_ref, o_ref):
        s3 = a3_ref[0, 0, :, 0:1] + a3_ref[1, 0, :, 0:1]
        o_ref[...] = jax.nn.sigmoid(s3 * invc_ref[...] + r3_ref[...])

    return pl.pallas_call(
        body,
        grid=(-(-n // BN),),
        in_specs=[pl.BlockSpec((2, 1, BN, 16), lambda i: (0, 0, i, 0)),
                  pl.BlockSpec((BN, 1), lambda i: (i, 0)),
                  pl.BlockSpec((BN, 1), lambda i: (i, 0))],
        out_specs=pl.BlockSpec((BN, 1), lambda i: (i, 0)),
        out_shape=jax.ShapeDtypeStruct((n, 1), F32),
    )(agg3p, invc, r3)


def kernel(x, edge_index, Wl1, Wr1, b1, Wl2, Wr2, b2, Wl3, Wr3, b3):
    n, d = x.shape
    e = edge_index.shape[1]

    grain = NS * ZR * 8
    npad = -(-(n + 1) // grain) * grain
    rps = npad // NS

    epg = NC * NS * CW * IB
    ep = -(-e // epg) * epg
    src = edge_index[0].astype(jnp.int32)
    dst = edge_index[1].astype(jnp.int32)
    trash = n + jnp.arange(ep - e, dtype=jnp.int32) % (npad - n)
    srcm = jnp.concatenate([src, jnp.zeros((ep - e,), jnp.int32)]).reshape(ep // CW, CW)
    dstm = jnp.concatenate([dst, trash]).reshape(ep // CW, CW)
    blocks = ep // (CW * IB)

    cntp = _sc_count(npad, rps, blocks, dstm)
    y1s, r1 = _tc1(n, d, npad, x, Wl1, Wr1)
    agg1p = _sc_agg_phased(npad, rps, blocks, 4, y1s, srcm, dstm)
    invc = _tc_cnt(npad, rps, cntp)
    y2s, r2 = _tc2(n, npad, agg1p, r1, invc, b1, Wl2, Wr2)
    agg2p = _sc_agg_phased(npad, rps, blocks, 2, y2s, srcm, dstm)
    y3s, r3 = _tc3(n, npad, agg2p, r2, invc, b2, Wl3, Wr3, b3)
    agg3p = _sc_agg_phased(npad, rps, blocks, 1, y3s, srcm, dstm)
    return _tc4(n, agg3p, invc, r3)
